# baseline (speedup 1.0000x reference)
import functools

import jax
import jax.numpy as jnp
from jax import lax
from jax.experimental import pallas as pl
from jax.experimental.pallas import tpu as pltpu
from jax.experimental.pallas import tpu_sc as plsc

_B = 4096
_L = 200
_DIM = 64
_N = _B * _L

_NC = 2
_NS = 16
_NW = _NC * _NS

_PER_W = _N // _NW
_GCH = 128
_KG = 4
_CH = _GCH * _KG
_NOUT = _PER_W // _CH
_NIDX = _PER_W // _GCH


def _emb_kernel(tok_hbm, table_hbm, out_hbm, idx_v, rows_v, gsem, osem):
    wid = lax.axis_index("s") * _NC + lax.axis_index("c")
    base = wid * _PER_W

    pltpu.sync_copy(tok_hbm.at[wid], idx_v)

    def body(i, carry):
        copies = []
        for jj in range(_KG):
            copies.append(
                pltpu.async_copy(
                    table_hbm.at[idx_v.at[i * _KG + jj]],
                    rows_v.at[pl.ds(jj * _GCH, _GCH)],
                    gsem,
                )
            )
        for c in copies:
            c.wait()
        pltpu.async_copy(
            rows_v, out_hbm.at[pl.ds(base + i * _CH, _CH)], osem
        ).wait()
        return carry

    lax.fori_loop(0, _NOUT, body, 0)


def _build():
    mesh = plsc.VectorSubcoreMesh(core_axis_name="c", subcore_axis_name="s")
    return functools.partial(
        pl.kernel,
        mesh=mesh,
        out_type=jax.ShapeDtypeStruct((_N, _DIM), jnp.float32),
        scratch_types=[
            pltpu.VMEM((_NIDX, _GCH), jnp.int32),
            pltpu.VMEM((_CH, _DIM), jnp.float32),
            pltpu.SemaphoreType.DMA,
            pltpu.SemaphoreType.DMA,
        ],
        compiler_params=pltpu.CompilerParams(use_tc_tiling_on_sc=False),
    )(_emb_kernel)


_lookup = _build()


@jax.jit
def kernel(token_id, table):
    tok = token_id.astype(jnp.int32).reshape(_NW, _NIDX, _GCH)
    out = _lookup(tok, table)
    return out.reshape(_B, _L, _DIM)


# speedup vs baseline: 1.0178x; 1.0178x over previous
import functools

import jax
import jax.numpy as jnp
from jax import lax
from jax.experimental import pallas as pl
from jax.experimental.pallas import tpu as pltpu
from jax.experimental.pallas import tpu_sc as plsc

_B = 4096
_L = 200
_DIM = 64
_N = _B * _L

_NC = 2
_NS = 16
_NW = _NC * _NS

_PER_W = _N // _NW
_GCH = 128
_KG = 4
_CH = _GCH * _KG
_NOUT = _PER_W // _CH
_NIDX = _PER_W // _GCH


def _emb_kernel(tok_hbm, table_hbm, out_hbm, idx_v, rows0, rows1, g0, g1, o0, o1):
    wid = lax.axis_index("s") * _NC + lax.axis_index("c")
    base = wid * _PER_W

    pltpu.sync_copy(tok_hbm.at[wid], idx_v)

    def fire(ci, buf, sem):
        for jj in range(_KG):
            pltpu.async_copy(
                table_hbm.at[idx_v.at[ci * _KG + jj]],
                buf.at[pl.ds(jj * _GCH, _GCH)],
                sem,
            )

    def wait_g(buf, sem):
        pltpu.make_async_copy(table_hbm.at[pl.ds(0, _CH)], buf, sem).wait()

    def out_start(ci, buf, sem):
        pltpu.async_copy(buf, out_hbm.at[pl.ds(base + ci * _CH, _CH)], sem)

    def wait_out(buf, sem):
        pltpu.make_async_copy(buf, out_hbm.at[pl.ds(base, _CH)], sem).wait()

    fire(0, rows0, g0)
    fire(1, rows1, g1)
    wait_g(rows0, g0)
    out_start(0, rows0, o0)
    wait_g(rows1, g1)
    out_start(1, rows1, o1)

    def body(p, carry):
        wait_out(rows0, o0)
        fire(2 * p, rows0, g0)
        wait_out(rows1, o1)
        fire(2 * p + 1, rows1, g1)
        wait_g(rows0, g0)
        out_start(2 * p, rows0, o0)
        wait_g(rows1, g1)
        out_start(2 * p + 1, rows1, o1)
        return carry

    lax.fori_loop(1, _NOUT // 2, body, 0)
    wait_out(rows0, o0)
    wait_out(rows1, o1)


def _build():
    mesh = plsc.VectorSubcoreMesh(core_axis_name="c", subcore_axis_name="s")
    return functools.partial(
        pl.kernel,
        mesh=mesh,
        out_type=jax.ShapeDtypeStruct((_N, _DIM), jnp.float32),
        scratch_types=[
            pltpu.VMEM((_NIDX, _GCH), jnp.int32),
            pltpu.VMEM((_CH, _DIM), jnp.float32),
            pltpu.VMEM((_CH, _DIM), jnp.float32),
            pltpu.SemaphoreType.DMA,
            pltpu.SemaphoreType.DMA,
            pltpu.SemaphoreType.DMA,
            pltpu.SemaphoreType.DMA,
        ],
        compiler_params=pltpu.CompilerParams(use_tc_tiling_on_sc=False),
    )(_emb_kernel)


_lookup = _build()


@jax.jit
def kernel(token_id, table):
    tok = token_id.astype(jnp.int32).reshape(_NW, _NIDX, _GCH)
    out = _lookup(tok, table)
    return out.reshape(_B, _L, _DIM)
